# Initial kernel scaffold; baseline (speedup 1.0000x reference)
import jax
import jax.numpy as jnp
from jax.experimental import pallas as pl
from jax.experimental.pallas import tpu as pltpu

BATCH = 16
SEQ = 512
IN_SIZE = 256
PROJ = 256
HID = 128
DENSE = 256
NUM_STEPS = 10
BETA = 0.95
THRESH = 1.0

FEAT = SEQ * HID
N_CHUNKS = 8
CHUNK = FEAT // N_CHUNKS


def _mem_kernel(x_ref, wp_ref, bp_ref, wg_ref, bg_ref, mem_ref, k_scr, bxw_scr):
    b = pl.program_id(0)

    @pl.when(b == 0)
    def _():
        k_scr[...] = jax.lax.dot_general(
            wp_ref[...], wg_ref[...],
            dimension_numbers=(((0,), (1,)), ((), ())),
            preferred_element_type=jnp.float32)
        bxw_scr[...] = jax.lax.dot_general(
            bp_ref[...], wg_ref[...],
            dimension_numbers=(((1,), (1,)), ((), ())),
            preferred_element_type=jnp.float32)

    xb = x_ref[0]
    xwb = jnp.dot(xb, k_scr[...], preferred_element_type=jnp.float32) + bxw_scr[...]

    t = jax.lax.broadcasted_iota(jnp.int32, (SEQ, 1), 0)
    end = (t == 0) | (t == SEQ - 1)
    dinv = jnp.where(end, 1.0 / jnp.sqrt(2.0), 1.0 / jnp.sqrt(3.0))
    zc = jnp.zeros((1, 1), jnp.float32)
    dinv_prev = jnp.concatenate([zc, dinv[:-1]], axis=0)
    dinv_next = jnp.concatenate([dinv[1:], zc], axis=0)
    cl = dinv_prev * dinv
    cr = dinv_next * dinv
    cs = dinv * dinv

    zrow = jnp.zeros((1, HID), jnp.float32)
    xw_prev = jnp.concatenate([zrow, xwb[:-1]], axis=0)
    xw_next = jnp.concatenate([xwb[1:], zrow], axis=0)
    cur = cl * xw_prev + cs * xwb + cr * xw_next + bg_ref[...]

    mem = cur
    cur_m1 = cur - THRESH
    for _ in range(NUM_STEPS - 1):
        mem = BETA * mem + jnp.where(mem > THRESH, cur_m1, cur)
    mem_ref[0] = mem


def _dense_kernel(mem_ref, wd_ref, bd_ref, wo_ref, bo_ref, out_ref, acc_ref):
    k = pl.program_id(0)

    @pl.when(k == 0)
    def _():
        acc_ref[...] = jnp.zeros_like(acc_ref)

    acc_ref[...] += jax.lax.dot_general(
        mem_ref[...], wd_ref[...],
        dimension_numbers=(((1,), (1,)), ((), ())),
        preferred_element_type=jnp.float32)

    @pl.when(k == N_CHUNKS - 1)
    def _():
        y = jnp.maximum(acc_ref[...] + bd_ref[...], 0.0)
        o = jax.lax.dot_general(
            y, wo_ref[...],
            dimension_numbers=(((1,), (1,)), ((), ())),
            preferred_element_type=jnp.float32)
        out_ref[...] = jax.nn.sigmoid(o + bo_ref[...])


def kernel(x, W_proj, b_proj, W_gcn, b_gcn, W_dense, b_dense, W_out, b_out):
    bp2 = b_proj.reshape(1, PROJ)
    bg2 = b_gcn.reshape(1, HID)
    bd2 = b_dense.reshape(1, DENSE)
    bo2 = b_out.reshape(1, 1)

    mem = pl.pallas_call(
        _mem_kernel,
        grid=(BATCH,),
        in_specs=[
            pl.BlockSpec((1, SEQ, IN_SIZE), lambda b: (b, 0, 0)),
            pl.BlockSpec((PROJ, IN_SIZE), lambda b: (0, 0)),
            pl.BlockSpec((1, PROJ), lambda b: (0, 0)),
            pl.BlockSpec((HID, PROJ), lambda b: (0, 0)),
            pl.BlockSpec((1, HID), lambda b: (0, 0)),
        ],
        out_specs=pl.BlockSpec((1, SEQ, HID), lambda b: (b, 0, 0)),
        out_shape=jax.ShapeDtypeStruct((BATCH, SEQ, HID), jnp.float32),
        scratch_shapes=[
            pltpu.VMEM((IN_SIZE, HID), jnp.float32),
            pltpu.VMEM((1, HID), jnp.float32),
        ],
    )(x, W_proj, bp2, W_gcn, bg2)

    mem2 = mem.reshape(BATCH, FEAT)

    out = pl.pallas_call(
        _dense_kernel,
        grid=(N_CHUNKS,),
        in_specs=[
            pl.BlockSpec((BATCH, CHUNK), lambda k: (0, k)),
            pl.BlockSpec((DENSE, CHUNK), lambda k: (0, k)),
            pl.BlockSpec((1, DENSE), lambda k: (0, 0)),
            pl.BlockSpec((1, DENSE), lambda k: (0, 0)),
            pl.BlockSpec((1, 1), lambda k: (0, 0)),
        ],
        out_specs=pl.BlockSpec((BATCH, 1), lambda k: (0, 0)),
        out_shape=jax.ShapeDtypeStruct((BATCH, 1), jnp.float32),
        scratch_shapes=[pltpu.VMEM((BATCH, DENSE), jnp.float32)],
    )(mem2, W_dense, bd2, W_out, bo2)

    return out


# speedup vs baseline: 5.3467x; 5.3467x over previous
import jax
import jax.numpy as jnp
from jax.experimental import pallas as pl
from jax.experimental.pallas import tpu as pltpu

BATCH = 16
SEQ = 512
IN_SIZE = 256
PROJ = 256
HID = 128
DENSE = 256
NUM_STEPS = 10
BETA = 0.95
THRESH = 1.0

FEAT = SEQ * HID
N_CHUNKS = 8
CHUNK = FEAT // N_CHUNKS


def _mem_kernel(x_ref, wp_ref, bp_ref, wg_ref, bg_ref, mem_ref):
    xb = x_ref[0]
    xp = jax.lax.dot_general(
        xb, wp_ref[...],
        dimension_numbers=(((1,), (1,)), ((), ())),
        preferred_element_type=jnp.float32) + bp_ref[...]
    xw = jax.lax.dot_general(
        xp, wg_ref[...],
        dimension_numbers=(((1,), (1,)), ((), ())),
        preferred_element_type=jnp.float32)

    t = jax.lax.broadcasted_iota(jnp.int32, (SEQ, 1), 0)
    end = (t == 0) | (t == SEQ - 1)
    dinv = jax.lax.rsqrt(jnp.where(end, 2.0, 3.0))
    zc = jnp.zeros((1, 1), jnp.float32)
    dinv_prev = jnp.concatenate([zc, dinv[:-1]], axis=0)
    dinv_next = jnp.concatenate([dinv[1:], zc], axis=0)
    cl = dinv_prev * dinv
    cr = dinv_next * dinv
    cs = dinv * dinv

    zrow = jnp.zeros((1, HID), jnp.float32)
    xw_prev = jnp.concatenate([zrow, xw[:-1]], axis=0)
    xw_next = jnp.concatenate([xw[1:], zrow], axis=0)
    cur = ((cl * xw_prev + cr * xw_next) + cs * xw) + bg_ref[...]

    mem = cur
    for _ in range(NUM_STEPS - 1):
        reset = jnp.where(mem > THRESH, jnp.float32(THRESH), jnp.float32(0.0))
        mem = BETA * mem + cur - reset
    mem_ref[0] = mem


def _dense_kernel(mem_ref, wd_ref, bd_ref, wo_ref, bo_ref, out_ref, acc_ref):
    k = pl.program_id(0)

    @pl.when(k == 0)
    def _():
        acc_ref[...] = jnp.zeros_like(acc_ref)

    acc_ref[...] += jax.lax.dot_general(
        mem_ref[...], wd_ref[...],
        dimension_numbers=(((1,), (1,)), ((), ())),
        preferred_element_type=jnp.float32)

    @pl.when(k == N_CHUNKS - 1)
    def _():
        y = jnp.maximum(acc_ref[...] + bd_ref[...], 0.0)
        o = jnp.sum(y * wo_ref[...], axis=1, keepdims=True)
        out_ref[...] = jax.nn.sigmoid(o + bo_ref[0, 0])


def kernel(x, W_proj, b_proj, W_gcn, b_gcn, W_dense, b_dense, W_out, b_out):
    bp2 = b_proj.reshape(1, PROJ)
    bg2 = b_gcn.reshape(1, HID)
    bd2 = b_dense.reshape(1, DENSE)
    bo2 = b_out.reshape(1, 1)

    mem = pl.pallas_call(
        _mem_kernel,
        grid=(BATCH,),
        in_specs=[
            pl.BlockSpec((1, SEQ, IN_SIZE), lambda b: (b, 0, 0)),
            pl.BlockSpec((PROJ, IN_SIZE), lambda b: (0, 0)),
            pl.BlockSpec((1, PROJ), lambda b: (0, 0)),
            pl.BlockSpec((HID, PROJ), lambda b: (0, 0)),
            pl.BlockSpec((1, HID), lambda b: (0, 0)),
        ],
        out_specs=pl.BlockSpec((1, SEQ, HID), lambda b: (b, 0, 0)),
        out_shape=jax.ShapeDtypeStruct((BATCH, SEQ, HID), jnp.float32),
    )(x, W_proj, bp2, W_gcn, bg2)

    mem2 = mem.reshape(BATCH, FEAT)

    out = pl.pallas_call(
        _dense_kernel,
        grid=(N_CHUNKS,),
        in_specs=[
            pl.BlockSpec((BATCH, CHUNK), lambda k: (0, k)),
            pl.BlockSpec((DENSE, CHUNK), lambda k: (0, k)),
            pl.BlockSpec((1, DENSE), lambda k: (0, 0)),
            pl.BlockSpec((1, DENSE), lambda k: (0, 0)),
            pl.BlockSpec(memory_space=pltpu.SMEM),
        ],
        out_specs=pl.BlockSpec((BATCH, 1), lambda k: (0, 0)),
        out_shape=jax.ShapeDtypeStruct((BATCH, 1), jnp.float32),
        scratch_shapes=[pltpu.VMEM((BATCH, DENSE), jnp.float32)],
    )(mem2, W_dense, bd2, W_out, bo2)

    return out
